# Initial kernel scaffold; baseline (speedup 1.0000x reference)
import jax
import jax.numpy as jnp
from jax.experimental import pallas as pl


def kernel(m_input, rbf, sbf, triplet_ids, reduce_to_ji, expand_to_kj, W_rbf1, W_rbf2, W_sbf1, W_sbf2, W_kj, b_kj, W_down, W_up, W_ji, b_ji, W_rb1a, b_rb1a, W_rb1b, b_rb1b, W_fbs, b_fbs, W_ra1a, b_ra1a, W_ra1b, b_ra1b, W_ra2a, b_ra2a, W_ra2b, b_ra2b):
    raise NotImplementedError("write your pallas kernel here")


# speedup vs baseline: 1.1488x; 1.1488x over previous
import functools

import jax
import jax.numpy as jnp
from jax import lax
from jax.experimental import pallas as pl
from jax.experimental.pallas import tpu as pltpu
from jax.experimental.pallas import tpu_sc as plsc

N_EDGES = 320000
N_TRIPLETS = 960000
EMBED = 128
ANGLE = 64

NC = 2
NS = 16
NW = NC * NS

E_CHUNK = 1250
NCH = N_EDGES // E_CHUNK
CH_PER_W = NCH // NW
T_BATCH = 128


def _swish(x):
    return x * jax.nn.sigmoid(x)


def _k1_body(m_ref, rbf_ref, wkj_ref, bkj_ref, wr1_ref, wr2_ref, wd_ref, out_ref):
    x = m_ref[...]
    h = jnp.dot(x, wkj_ref[...], preferred_element_type=jnp.float32) + bkj_ref[...]
    h = _swish(h)
    r = jnp.dot(rbf_ref[...], wr1_ref[...], preferred_element_type=jnp.float32)
    r = jnp.dot(r, wr2_ref[...], preferred_element_type=jnp.float32)
    h = h * r
    g = jnp.dot(h, wd_ref[...], preferred_element_type=jnp.float32)
    out_ref[...] = _swish(g)


def _k1(m_input, rbf, W_kj, b_kj, W_rbf1, W_rbf2, W_down):
    B = 1280
    grid = (N_EDGES // B,)
    full = lambda shape: pl.BlockSpec(shape, lambda i: (0, 0))
    return pl.pallas_call(
        _k1_body,
        grid=grid,
        in_specs=[
            pl.BlockSpec((B, EMBED), lambda i: (i, 0)),
            pl.BlockSpec((B, 6), lambda i: (i, 0)),
            full(W_kj.shape),
            full((1, EMBED)),
            full(W_rbf1.shape),
            full(W_rbf2.shape),
            full(W_down.shape),
        ],
        out_specs=pl.BlockSpec((B, ANGLE), lambda i: (i, 0)),
        out_shape=jax.ShapeDtypeStruct((N_EDGES, ANGLE), jnp.float32),
    )(m_input, rbf, W_kj, b_kj.reshape(1, EMBED), W_rbf1, W_rbf2, W_down)


def _k2_body(sbf_ref, w1_ref, w2_ref, out_ref):
    s = jnp.dot(sbf_ref[...], w1_ref[...], preferred_element_type=jnp.float32)
    out_ref[...] = jnp.dot(s, w2_ref[...], preferred_element_type=jnp.float32)


def _k2(sbf, W_sbf1, W_sbf2):
    B = 1280
    grid = (N_TRIPLETS // B,)
    return pl.pallas_call(
        _k2_body,
        grid=grid,
        in_specs=[
            pl.BlockSpec((B, 42), lambda i: (i, 0)),
            pl.BlockSpec(W_sbf1.shape, lambda i: (0, 0)),
            pl.BlockSpec(W_sbf2.shape, lambda i: (0, 0)),
        ],
        out_specs=pl.BlockSpec((B, ANGLE), lambda i: (i, 0)),
        out_shape=jax.ShapeDtypeStruct((N_TRIPLETS, ANGLE), jnp.float32),
    )(sbf, W_sbf1, W_sbf2)


def _sc_body(m_ang_hbm, sb_hbm, expand_hbm, reduce_hbm, t0_hbm, nb_hbm,
             agg_hbm, acc_v, idx_e, idx_r, rows_v, sbv_v, t0_v, nb_v, sem):
    wid = lax.axis_index("s") * NC + lax.axis_index("c")
    pltpu.sync_copy(t0_hbm, t0_v)
    pltpu.sync_copy(nb_hbm, nb_v)
    zeros16 = jnp.zeros((16,), jnp.float32)
    t0g = t0_v[pl.ds(wid * CH_PER_W, 16)]
    nbg = nb_v[pl.ds(wid * CH_PER_W, 16)]

    for ci in range(CH_PER_W):
        c = wid * CH_PER_W + ci
        e0 = c * E_CHUNK

        def zero_body(i, _):
            acc_v[pl.ds(i * 16, 16)] = zeros16
            return 0
        lax.fori_loop(0, E_CHUNK * ANGLE // 16, zero_body, 0)

        t0c = t0g[ci]
        nbc = nbg[ci]

        def batch_body(k, _):
            t = pl.multiple_of(t0c + k * T_BATCH, T_BATCH)
            pltpu.sync_copy(expand_hbm.at[pl.ds(t, T_BATCH)], idx_e)
            pltpu.sync_copy(reduce_hbm.at[pl.ds(t, T_BATCH)], idx_r)
            gather = pltpu.async_copy(m_ang_hbm.at[idx_e], rows_v, sem)
            pltpu.sync_copy(sb_hbm.at[pl.ds(t, T_BATCH)], sbv_v)
            gather.wait()

            def group_body(g, _):
                rv = idx_r[pl.ds(g * 16, 16)] - e0
                okv = jnp.logical_and(rv >= 0, rv < E_CHUNK)
                wv = jnp.where(okv, jnp.float32(1.0), jnp.float32(0.0))
                basev = jnp.clip(rv, 0, E_CHUNK - 1) * ANGLE
                for j in range(16):
                    jj = g * 16 + j
                    base = basev[j]
                    w = wv[j]
                    for f in range(ANGLE // 16):
                        v = rows_v[jj, pl.ds(16 * f, 16)] * sbv_v[jj, pl.ds(16 * f, 16)]
                        plsc.addupdate(acc_v.at[pl.ds(base + 16 * f, 16)], v * w)
                return 0
            lax.fori_loop(0, T_BATCH // 16, group_body, 0)
            return 0
        lax.fori_loop(0, nbc, batch_body, 0)

        pltpu.sync_copy(acc_v, agg_hbm.at[pl.ds(e0 * ANGLE, E_CHUNK * ANGLE)])


def _sc_segment(m_ang, sb, expand_to_kj, reduce_to_ji, t0, nb):
    mesh = plsc.VectorSubcoreMesh(core_axis_name="c", subcore_axis_name="s",
                                  num_cores=NC, num_subcores=NS)
    f = pl.kernel(
        _sc_body,
        out_type=jax.ShapeDtypeStruct((N_EDGES * ANGLE,), jnp.float32),
        mesh=mesh,
        scratch_types=[
            pltpu.VMEM((E_CHUNK * ANGLE,), jnp.float32),
            pltpu.VMEM((T_BATCH,), jnp.int32),
            pltpu.VMEM((T_BATCH,), jnp.int32),
            pltpu.VMEM((T_BATCH, ANGLE), jnp.float32),
            pltpu.VMEM((T_BATCH, ANGLE), jnp.float32),
            pltpu.VMEM((NCH + 16,), jnp.int32),
            pltpu.VMEM((NCH + 16,), jnp.int32),
            pltpu.SemaphoreType.DMA,
        ],
        compiler_params=pltpu.CompilerParams(use_tc_tiling_on_sc=False),
    )
    return f(m_ang, sb, expand_to_kj, reduce_to_ji, t0, nb)


def _k3_body(m_ref, agg_ref, wup_ref, wji_ref, bji_ref,
             wrb1a_ref, brb1a_ref, wrb1b_ref, brb1b_ref,
             wfbs_ref, bfbs_ref,
             wra1a_ref, bra1a_ref, wra1b_ref, bra1b_ref,
             wra2a_ref, bra2a_ref, wra2b_ref, bra2b_ref, out_ref):
    dot = lambda a, b: jnp.dot(a, b, preferred_element_type=jnp.float32)
    x = m_ref[...]
    prop = _swish(dot(agg_ref[...], wup_ref[...]))
    m_ji = _swish(dot(x, wji_ref[...]) + bji_ref[...])
    mc = m_ji + prop
    t = _swish(dot(mc, wrb1a_ref[...]) + brb1a_ref[...])
    mc = mc + _swish(dot(t, wrb1b_ref[...]) + brb1b_ref[...])
    mc = _swish(dot(mc, wfbs_ref[...]) + bfbs_ref[...])
    out = mc + x
    t = _swish(dot(out, wra1a_ref[...]) + bra1a_ref[...])
    out = out + _swish(dot(t, wra1b_ref[...]) + bra1b_ref[...])
    t = _swish(dot(out, wra2a_ref[...]) + bra2a_ref[...])
    out_ref[...] = out + _swish(dot(t, wra2b_ref[...]) + bra2b_ref[...])


def _k3(m_input, agg, W_up, W_ji, b_ji, W_rb1a, b_rb1a, W_rb1b, b_rb1b,
        W_fbs, b_fbs, W_ra1a, b_ra1a, W_ra1b, b_ra1b, W_ra2a, b_ra2a,
        W_ra2b, b_ra2b):
    B = 1280
    grid = (N_EDGES // B,)
    full = lambda shape: pl.BlockSpec(shape, lambda i: (0, 0))
    row = lambda b: b.reshape(1, EMBED)
    return pl.pallas_call(
        _k3_body,
        grid=grid,
        in_specs=[
            pl.BlockSpec((B, EMBED), lambda i: (i, 0)),
            pl.BlockSpec((B, ANGLE), lambda i: (i, 0)),
            full(W_up.shape), full(W_ji.shape), full((1, EMBED)),
            full(W_rb1a.shape), full((1, EMBED)), full(W_rb1b.shape), full((1, EMBED)),
            full(W_fbs.shape), full((1, EMBED)),
            full(W_ra1a.shape), full((1, EMBED)), full(W_ra1b.shape), full((1, EMBED)),
            full(W_ra2a.shape), full((1, EMBED)), full(W_ra2b.shape), full((1, EMBED)),
        ],
        out_specs=pl.BlockSpec((B, EMBED), lambda i: (i, 0)),
        out_shape=jax.ShapeDtypeStruct((N_EDGES, EMBED), jnp.float32),
    )(m_input, agg, W_up, W_ji, row(b_ji),
      W_rb1a, row(b_rb1a), W_rb1b, row(b_rb1b),
      W_fbs, row(b_fbs),
      W_ra1a, row(b_ra1a), W_ra1b, row(b_ra1b),
      W_ra2a, row(b_ra2a), W_ra2b, row(b_ra2b))


def kernel(m_input, rbf, sbf, triplet_ids, reduce_to_ji, expand_to_kj,
           W_rbf1, W_rbf2, W_sbf1, W_sbf2, W_kj, b_kj, W_down, W_up,
           W_ji, b_ji, W_rb1a, b_rb1a, W_rb1b, b_rb1b, W_fbs, b_fbs,
           W_ra1a, b_ra1a, W_ra1b, b_ra1b, W_ra2a, b_ra2a, W_ra2b, b_ra2b):
    m_ang = _k1(m_input, rbf, W_kj, b_kj, W_rbf1, W_rbf2, W_down)
    sb = _k2(sbf, W_sbf1, W_sbf2)

    bounds = jnp.arange(NCH + 1, dtype=jnp.int32) * E_CHUNK
    pos = jnp.searchsorted(reduce_to_ji, bounds, side="left").astype(jnp.int32)
    starts, ends = pos[:-1], pos[1:]
    lo = starts // T_BATCH
    hi = (ends + T_BATCH - 1) // T_BATCH
    pad = jnp.zeros((16,), jnp.int32)
    t0 = jnp.concatenate([lo * T_BATCH, pad])
    nb = jnp.concatenate([jnp.where(ends > starts, hi - lo, 0).astype(jnp.int32), pad])

    agg = _sc_segment(m_ang, sb, expand_to_kj, reduce_to_ji, t0, nb)
    agg = agg.reshape(N_EDGES, ANGLE)

    return _k3(m_input, agg, W_up, W_ji, b_ji, W_rb1a, b_rb1a, W_rb1b, b_rb1b,
               W_fbs, b_fbs, W_ra1a, b_ra1a, W_ra1b, b_ra1b,
               W_ra2a, b_ra2a, W_ra2b, b_ra2b)


# speedup vs baseline: 1.7564x; 1.5289x over previous
import jax
import jax.numpy as jnp
from jax import lax
from jax.experimental import pallas as pl
from jax.experimental.pallas import tpu as pltpu
from jax.experimental.pallas import tpu_sc as plsc

N_EDGES = 320000
N_TRIPLETS = 960000
EMBED = 128
ANGLE = 64

NC = 2
NS = 16
NW = NC * NS

E_CHUNK = 512
NCH = N_EDGES // E_CHUNK
CH_PER_W = 20
T_BATCH = 128
NB_T = N_TRIPLETS // T_BATCH


def _swish(x):
    return x * (0.5 * jnp.tanh(0.5 * x) + 0.5)


def _k1_body(m_ref, rbft_ref, wkj_ref, bkj_ref, wr_ref, wd_ref, out_ref):
    x = m_ref[...]
    h = jnp.dot(x, wkj_ref[...], preferred_element_type=jnp.float32) + bkj_ref[...]
    h = _swish(h)
    r = lax.dot_general(rbft_ref[...], wr_ref[...],
                        (((0,), (0,)), ((), ())),
                        preferred_element_type=jnp.float32)
    h = h * r
    g = jnp.dot(h, wd_ref[...], preferred_element_type=jnp.float32)
    out_ref[...] = _swish(g)


def _k1(m_input, rbfT, W_kj, b_kj, W_rbf, W_down_pad):
    B = 1280
    grid = (N_EDGES // B,)
    full = lambda shape: pl.BlockSpec(shape, lambda i: (0, 0))
    return pl.pallas_call(
        _k1_body,
        grid=grid,
        in_specs=[
            pl.BlockSpec((B, EMBED), lambda i: (i, 0)),
            pl.BlockSpec((6, B), lambda i: (0, i)),
            full(W_kj.shape),
            full((1, EMBED)),
            full(W_rbf.shape),
            full(W_down_pad.shape),
        ],
        out_specs=pl.BlockSpec((B, EMBED), lambda i: (i, 0)),
        out_shape=jax.ShapeDtypeStruct((N_EDGES, EMBED), jnp.float32),
    )(m_input, rbfT, W_kj, b_kj.reshape(1, EMBED), W_rbf, W_down_pad)


def _k2_body(sbft_ref, w_ref, out_ref):
    out_ref[...] = lax.dot_general(sbft_ref[...], w_ref[...],
                                   (((0,), (0,)), ((), ())),
                                   preferred_element_type=jnp.float32)


def _k2(sbfT, W_sbf_pad):
    B = 1920
    grid = (N_TRIPLETS // B,)
    return pl.pallas_call(
        _k2_body,
        grid=grid,
        in_specs=[
            pl.BlockSpec((42, B), lambda i: (0, i)),
            pl.BlockSpec(W_sbf_pad.shape, lambda i: (0, 0)),
        ],
        out_specs=pl.BlockSpec((B, 128), lambda i: (i, 0)),
        out_shape=jax.ShapeDtypeStruct((N_TRIPLETS, 128), jnp.float32),
    )(sbfT, W_sbf_pad)


def _sc_body(m_ang_hbm, sb_hbm, expand_hbm, reduce_hbm, t0_hbm, nb_hbm,
             agg_hbm, acc_v, idx_e, idx_r, rows_v, sbv_v, t0_v, nb_v, sem):
    wid = lax.axis_index("s") * NC + lax.axis_index("c")
    pltpu.sync_copy(t0_hbm, t0_v)
    pltpu.sync_copy(nb_hbm, nb_v)
    zeros16 = jnp.zeros((16,), jnp.float32)
    t0a = t0_v[pl.ds(wid * 32, 16)]
    t0b = t0_v[pl.ds(wid * 32 + 16, 16)]
    nba = nb_v[pl.ds(wid * 32, 16)]
    nbb = nb_v[pl.ds(wid * 32 + 16, 16)]

    for ci in range(CH_PER_W):
        c = wid * CH_PER_W + ci
        e0 = c * E_CHUNK
        t0c = t0a[ci] if ci < 16 else t0b[ci - 16]
        nbc = nba[ci] if ci < 16 else nbb[ci - 16]

        def zero_body(r, _):
            for f in range(8):
                acc_v[r, pl.ds(16 * f, 16)] = zeros16
            return 0
        lax.fori_loop(0, E_CHUNK + 1, zero_body, 0)

        def batch_body(k, _):
            t = pl.multiple_of(t0c + k * T_BATCH, T_BATCH)
            pltpu.sync_copy(expand_hbm.at[pl.ds(t, T_BATCH)], idx_e)
            pltpu.sync_copy(reduce_hbm.at[pl.ds(t, T_BATCH)], idx_r)
            gather = pltpu.async_copy(m_ang_hbm.at[idx_e], rows_v, sem)
            pltpu.sync_copy(sb_hbm.at[pl.ds(t, T_BATCH)], sbv_v)
            gather.wait()

            def group_body(g, _):
                rv = idx_r[pl.ds(g * 16, 16)] - e0
                okv = jnp.logical_and(rv >= 0, rv < E_CHUNK)
                rowv = jnp.where(okv, rv, E_CHUNK)
                for j in range(16):
                    jj = g * 16 + j
                    row = rowv[j]
                    for f in range(ANGLE // 16):
                        v = rows_v[jj, pl.ds(16 * f, 16)] * sbv_v[jj, pl.ds(16 * f, 16)]
                        plsc.addupdate(acc_v.at[row, pl.ds(16 * f, 16)], v)
                return 0
            lax.fori_loop(0, T_BATCH // 16, group_body, 0)
            return 0
        lax.fori_loop(0, nbc, batch_body, 0)

        @pl.when(c < NCH)
        def _flush():
            pltpu.sync_copy(acc_v.at[pl.ds(0, E_CHUNK)],
                            agg_hbm.at[pl.ds(e0, E_CHUNK)])


def _sc_segment(m_ang_pad, sb_pad, expand_to_kj, reduce_to_ji, t0m, nbm):
    mesh = plsc.VectorSubcoreMesh(core_axis_name="c", subcore_axis_name="s",
                                  num_cores=NC, num_subcores=NS)
    f = pl.kernel(
        _sc_body,
        out_type=jax.ShapeDtypeStruct((N_EDGES, 128), jnp.float32),
        mesh=mesh,
        scratch_types=[
            pltpu.VMEM((E_CHUNK + 1, 128), jnp.float32),
            pltpu.VMEM((T_BATCH,), jnp.int32),
            pltpu.VMEM((T_BATCH,), jnp.int32),
            pltpu.VMEM((T_BATCH, 128), jnp.float32),
            pltpu.VMEM((T_BATCH, 128), jnp.float32),
            pltpu.VMEM((NW * 32,), jnp.int32),
            pltpu.VMEM((NW * 32,), jnp.int32),
            pltpu.SemaphoreType.DMA,
        ],
        compiler_params=pltpu.CompilerParams(use_tc_tiling_on_sc=True),
    )
    return f(m_ang_pad, sb_pad, expand_to_kj, reduce_to_ji, t0m, nbm)


def _k3_body(m_ref, agg_ref, wup_ref, wji_ref, bji_ref,
             wrb1a_ref, brb1a_ref, wrb1b_ref, brb1b_ref,
             wfbs_ref, bfbs_ref,
             wra1a_ref, bra1a_ref, wra1b_ref, bra1b_ref,
             wra2a_ref, bra2a_ref, wra2b_ref, bra2b_ref, out_ref):
    dot = lambda a, b: jnp.dot(a, b, preferred_element_type=jnp.float32)
    x = m_ref[...]
    prop = _swish(dot(agg_ref[...], wup_ref[...]))
    m_ji = _swish(dot(x, wji_ref[...]) + bji_ref[...])
    mc = m_ji + prop
    t = _swish(dot(mc, wrb1a_ref[...]) + brb1a_ref[...])
    mc = mc + _swish(dot(t, wrb1b_ref[...]) + brb1b_ref[...])
    mc = _swish(dot(mc, wfbs_ref[...]) + bfbs_ref[...])
    out = mc + x
    t = _swish(dot(out, wra1a_ref[...]) + bra1a_ref[...])
    out = out + _swish(dot(t, wra1b_ref[...]) + bra1b_ref[...])
    t = _swish(dot(out, wra2a_ref[...]) + bra2a_ref[...])
    out_ref[...] = out + _swish(dot(t, wra2b_ref[...]) + bra2b_ref[...])


def _k3(m_input, agg_pad, W_up_pad, W_ji, b_ji, W_rb1a, b_rb1a, W_rb1b, b_rb1b,
        W_fbs, b_fbs, W_ra1a, b_ra1a, W_ra1b, b_ra1b, W_ra2a, b_ra2a,
        W_ra2b, b_ra2b):
    B = 1280
    grid = (N_EDGES // B,)
    full = lambda shape: pl.BlockSpec(shape, lambda i: (0, 0))
    row = lambda b: b.reshape(1, EMBED)
    return pl.pallas_call(
        _k3_body,
        grid=grid,
        in_specs=[
            pl.BlockSpec((B, EMBED), lambda i: (i, 0)),
            pl.BlockSpec((B, 128), lambda i: (i, 0)),
            full(W_up_pad.shape), full(W_ji.shape), full((1, EMBED)),
            full(W_rb1a.shape), full((1, EMBED)), full(W_rb1b.shape), full((1, EMBED)),
            full(W_fbs.shape), full((1, EMBED)),
            full(W_ra1a.shape), full((1, EMBED)), full(W_ra1b.shape), full((1, EMBED)),
            full(W_ra2a.shape), full((1, EMBED)), full(W_ra2b.shape), full((1, EMBED)),
        ],
        out_specs=pl.BlockSpec((B, EMBED), lambda i: (i, 0)),
        out_shape=jax.ShapeDtypeStruct((N_EDGES, EMBED), jnp.float32),
    )(m_input, agg_pad, W_up_pad, W_ji, row(b_ji),
      W_rb1a, row(b_rb1a), W_rb1b, row(b_rb1b),
      W_fbs, row(b_fbs),
      W_ra1a, row(b_ra1a), W_ra1b, row(b_ra1b),
      W_ra2a, row(b_ra2a), W_ra2b, row(b_ra2b))


def kernel(m_input, rbf, sbf, triplet_ids, reduce_to_ji, expand_to_kj,
           W_rbf1, W_rbf2, W_sbf1, W_sbf2, W_kj, b_kj, W_down, W_up,
           W_ji, b_ji, W_rb1a, b_rb1a, W_rb1b, b_rb1b, W_fbs, b_fbs,
           W_ra1a, b_ra1a, W_ra1b, b_ra1b, W_ra2a, b_ra2a, W_ra2b, b_ra2b):
    W_rbf = W_rbf1 @ W_rbf2
    W_sbf_pad = jnp.pad(W_sbf1 @ W_sbf2, ((0, 0), (0, 64)))
    W_down_pad = jnp.pad(W_down, ((0, 0), (0, 64)))
    W_up_pad = jnp.pad(W_up, ((0, 64), (0, 0)))

    m_ang_pad = _k1(m_input, rbf.T, W_kj, b_kj, W_rbf, W_down_pad)
    sb_pad = _k2(sbf.T, W_sbf_pad)

    sv_start = lax.slice(reduce_to_ji, (0,), (N_TRIPLETS,), (T_BATCH,))
    sv_end = lax.slice(reduce_to_ji, (T_BATCH - 1,), (N_TRIPLETS,), (T_BATCH,))
    e0s = jnp.arange(NCH, dtype=jnp.int32) * E_CHUNK
    lo = jnp.sum(sv_end[None, :] < e0s[:, None], axis=1, dtype=jnp.int32)
    hi = jnp.sum(sv_start[None, :] < (e0s[:, None] + E_CHUNK), axis=1,
                 dtype=jnp.int32)
    t0 = lo * T_BATCH
    nb = hi - lo
    pack = lambda v: jnp.pad(
        jnp.pad(v, (0, NW * CH_PER_W - NCH)).reshape(NW, CH_PER_W),
        ((0, 0), (0, 32 - CH_PER_W))).reshape(-1)

    agg_pad = _sc_segment(m_ang_pad, sb_pad, expand_to_kj, reduce_to_ji,
                          pack(t0), pack(nb))

    return _k3(m_input, agg_pad, W_up_pad, W_ji, b_ji, W_rb1a, b_rb1a,
               W_rb1b, b_rb1b, W_fbs, b_fbs, W_ra1a, b_ra1a, W_ra1b, b_ra1b,
               W_ra2a, b_ra2a, W_ra2b, b_ra2b)


# speedup vs baseline: 1.9987x; 1.1380x over previous
import jax
import jax.numpy as jnp
from jax import lax
from jax.experimental import pallas as pl
from jax.experimental.pallas import tpu as pltpu
from jax.experimental.pallas import tpu_sc as plsc

N_EDGES = 320000
N_TRIPLETS = 960000
EMBED = 128
ANGLE = 64

NC = 2
NS = 16
NW = NC * NS

E_CHUNK = 320
NCH = N_EDGES // E_CHUNK
CH_PER_W = 32
T_BATCH = 128
NB_T = N_TRIPLETS // T_BATCH


def _swish(x):
    return x * (0.5 * jnp.tanh(0.5 * x) + 0.5)


def _k1_body(m_ref, rbft_ref, wkj_ref, bkj_ref, wr_ref, wd_ref, out_ref):
    x = m_ref[...]
    h = jnp.dot(x, wkj_ref[...], preferred_element_type=jnp.float32) + bkj_ref[...]
    h = _swish(h)
    r = lax.dot_general(rbft_ref[...], wr_ref[...],
                        (((0,), (0,)), ((), ())),
                        preferred_element_type=jnp.float32)
    h = h * r
    g = jnp.dot(h, wd_ref[...], preferred_element_type=jnp.float32)
    out_ref[...] = _swish(g)


def _k1(m_input, rbfT, W_kj, b_kj, W_rbf, W_down_pad):
    B = 1280
    grid = (N_EDGES // B,)
    full = lambda shape: pl.BlockSpec(shape, lambda i: (0, 0))
    return pl.pallas_call(
        _k1_body,
        grid=grid,
        in_specs=[
            pl.BlockSpec((B, EMBED), lambda i: (i, 0)),
            pl.BlockSpec((6, B), lambda i: (0, i)),
            full(W_kj.shape),
            full((1, EMBED)),
            full(W_rbf.shape),
            full(W_down_pad.shape),
        ],
        out_specs=pl.BlockSpec((B, EMBED), lambda i: (i, 0)),
        out_shape=jax.ShapeDtypeStruct((N_EDGES, EMBED), jnp.float32),
    )(m_input, rbfT, W_kj, b_kj.reshape(1, EMBED), W_rbf, W_down_pad)


def _k2_body(sbft_ref, w_ref, out_ref):
    out_ref[...] = lax.dot_general(sbft_ref[...], w_ref[...],
                                   (((0,), (0,)), ((), ())),
                                   preferred_element_type=jnp.float32)


def _k2(sbfT, W_sbf_pad):
    B = 1920
    grid = (N_TRIPLETS // B,)
    return pl.pallas_call(
        _k2_body,
        grid=grid,
        in_specs=[
            pl.BlockSpec((42, B), lambda i: (0, i)),
            pl.BlockSpec(W_sbf_pad.shape, lambda i: (0, 0)),
        ],
        out_specs=pl.BlockSpec((B, 128), lambda i: (i, 0)),
        out_shape=jax.ShapeDtypeStruct((N_TRIPLETS, 128), jnp.float32),
    )(sbfT, W_sbf_pad)


def _sc_body(m_ang_hbm, sb_hbm, expand_hbm, reduce_hbm, t0_hbm, nb_hbm,
             agg_hbm, acc_v,
             idx_e_a, idx_r_a, rows_a, sbv_a,
             idx_e_b, idx_r_b, rows_b, sbv_b,
             t0_v, nb_v, t0_s, nb_s, sem_la, sem_lb, sem_ga, sem_gb):
    wid = lax.axis_index("s") * NC + lax.axis_index("c")
    pltpu.sync_copy(t0_hbm.at[pl.ds(wid * 32, 32)], t0_v)
    pltpu.sync_copy(nb_hbm.at[pl.ds(wid * 32, 32)], nb_v)
    zeros16 = jnp.zeros((16,), jnp.float32)
    for g in range(2):
        tv = t0_v[pl.ds(g * 16, 16)]
        nv = nb_v[pl.ds(g * 16, 16)]
        for j in range(16):
            t0_s[g * 16 + j] = tv[j]
            nb_s[g * 16 + j] = nv[j]

    bufs = ((idx_e_a, idx_r_a, rows_a, sbv_a, sem_la, sem_ga),
            (idx_e_b, idx_r_b, rows_b, sbv_b, sem_lb, sem_gb))

    def issue_loads(t, bi):
        idx_e, idx_r, _, sbv, sem_l, _ = bufs[bi]
        pltpu.async_copy(expand_hbm.at[pl.ds(t, T_BATCH)], idx_e, sem_l)
        pltpu.async_copy(reduce_hbm.at[pl.ds(t, T_BATCH)], idx_r, sem_l)
        pltpu.async_copy(sb_hbm.at[pl.ds(t, T_BATCH)], sbv, sem_l)

    def wait_loads(bi):
        idx_e, idx_r, _, sbv, sem_l, _ = bufs[bi]
        pltpu.make_async_copy(expand_hbm.at[pl.ds(0, T_BATCH)], idx_e, sem_l).wait()
        pltpu.make_async_copy(reduce_hbm.at[pl.ds(0, T_BATCH)], idx_r, sem_l).wait()
        pltpu.make_async_copy(sb_hbm.at[pl.ds(0, T_BATCH)], sbv, sem_l).wait()

    def issue_gather(bi):
        idx_e, _, rows, _, _, sem_g = bufs[bi]
        pltpu.async_copy(m_ang_hbm.at[idx_e], rows, sem_g)

    def wait_gather(bi):
        idx_e, _, rows, _, _, sem_g = bufs[bi]
        pltpu.make_async_copy(m_ang_hbm.at[idx_e], rows, sem_g).wait()

    def chunk_body(ci, _carry):
        c = wid * CH_PER_W + ci
        e0 = c * E_CHUNK
        t0c = t0_s[ci]
        nbc = nb_s[ci]

        def zero_body(r, _):
            for f in range(8):
                acc_v[r, pl.ds(16 * f, 16)] = zeros16
            return 0
        lax.fori_loop(0, E_CHUNK + 1, zero_body, 0)

        def tt(k):
            return pl.multiple_of(t0c + k * T_BATCH, T_BATCH)

        def compute(bi):
            idx_e, idx_r, rows, sbv, _, _ = bufs[bi]

            def group_body(g, _):
                rv = idx_r[pl.ds(g * 16, 16)] - e0
                okv = jnp.logical_and(rv >= 0, rv < E_CHUNK)
                rowv = jnp.where(okv, rv, E_CHUNK)
                for j in range(16):
                    jj = g * 16 + j
                    row = rowv[j]
                    for f in range(ANGLE // 16):
                        v = rows[jj, pl.ds(16 * f, 16)] * sbv[jj, pl.ds(16 * f, 16)]
                        plsc.addupdate(acc_v.at[row, pl.ds(16 * f, 16)], v)
                return 0
            lax.fori_loop(0, T_BATCH // 16, group_body, 0)

        @pl.when(nbc > 0)
        def _prologue():
            issue_loads(tt(0), 0)

        def pair_body(m, _):
            k0 = 2 * m
            k1 = k0 + 1

            @pl.when(k0 < nbc)
            def _half0():
                wait_loads(0)
                issue_gather(0)

                @pl.when(k0 > 0)
                def _fin_prev():
                    wait_gather(1)
                    compute(1)

                @pl.when(k1 < nbc)
                def _next():
                    issue_loads(tt(k1), 1)

            @pl.when(k1 < nbc)
            def _half1():
                wait_loads(1)
                issue_gather(1)
                wait_gather(0)
                compute(0)

                @pl.when(k1 + 1 < nbc)
                def _next():
                    issue_loads(tt(k1 + 1), 0)
            return 0
        lax.fori_loop(0, (nbc + 1) // 2, pair_body, 0)

        @pl.when(jnp.logical_and(nbc > 0, (nbc & 1) == 1))
        def _epi_even():
            wait_gather(0)
            compute(0)

        @pl.when(jnp.logical_and(nbc > 0, (nbc & 1) == 0))
        def _epi_odd():
            wait_gather(1)
            compute(1)

        @pl.when(c < NCH)
        def _flush():
            pltpu.sync_copy(acc_v.at[pl.ds(0, E_CHUNK)],
                            agg_hbm.at[pl.ds(e0, E_CHUNK)])
        return 0

    lax.fori_loop(0, CH_PER_W, chunk_body, 0)


def _sc_segment(m_ang_pad, sb_pad, expand_to_kj, reduce_to_ji, t0m, nbm):
    mesh = plsc.VectorSubcoreMesh(core_axis_name="c", subcore_axis_name="s",
                                  num_cores=NC, num_subcores=NS)
    f = pl.kernel(
        _sc_body,
        out_type=jax.ShapeDtypeStruct((N_EDGES, 128), jnp.float32),
        mesh=mesh,
        scratch_types=[
            pltpu.VMEM((E_CHUNK + 1, 128), jnp.float32),
            pltpu.VMEM((T_BATCH,), jnp.int32),
            pltpu.VMEM((T_BATCH,), jnp.int32),
            pltpu.VMEM((T_BATCH, 128), jnp.float32),
            pltpu.VMEM((T_BATCH, 128), jnp.float32),
            pltpu.VMEM((T_BATCH,), jnp.int32),
            pltpu.VMEM((T_BATCH,), jnp.int32),
            pltpu.VMEM((T_BATCH, 128), jnp.float32),
            pltpu.VMEM((T_BATCH, 128), jnp.float32),
            pltpu.VMEM((32,), jnp.int32),
            pltpu.VMEM((32,), jnp.int32),
            pltpu.SMEM((32,), jnp.int32),
            pltpu.SMEM((32,), jnp.int32),
            pltpu.SemaphoreType.DMA,
            pltpu.SemaphoreType.DMA,
            pltpu.SemaphoreType.DMA,
            pltpu.SemaphoreType.DMA,
        ],
        compiler_params=pltpu.CompilerParams(use_tc_tiling_on_sc=True),
    )
    return f(m_ang_pad, sb_pad, expand_to_kj, reduce_to_ji, t0m, nbm)


def _k3_body(m_ref, agg_ref, wup_ref, wji_ref, bji_ref,
             wrb1a_ref, brb1a_ref, wrb1b_ref, brb1b_ref,
             wfbs_ref, bfbs_ref,
             wra1a_ref, bra1a_ref, wra1b_ref, bra1b_ref,
             wra2a_ref, bra2a_ref, wra2b_ref, bra2b_ref, out_ref):
    dot = lambda a, b: jnp.dot(a, b, preferred_element_type=jnp.float32)
    x = m_ref[...]
    prop = _swish(dot(agg_ref[...], wup_ref[...]))
    m_ji = _swish(dot(x, wji_ref[...]) + bji_ref[...])
    mc = m_ji + prop
    t = _swish(dot(mc, wrb1a_ref[...]) + brb1a_ref[...])
    mc = mc + _swish(dot(t, wrb1b_ref[...]) + brb1b_ref[...])
    mc = _swish(dot(mc, wfbs_ref[...]) + bfbs_ref[...])
    out = mc + x
    t = _swish(dot(out, wra1a_ref[...]) + bra1a_ref[...])
    out = out + _swish(dot(t, wra1b_ref[...]) + bra1b_ref[...])
    t = _swish(dot(out, wra2a_ref[...]) + bra2a_ref[...])
    out_ref[...] = out + _swish(dot(t, wra2b_ref[...]) + bra2b_ref[...])


def _k3(m_input, agg_pad, W_up_pad, W_ji, b_ji, W_rb1a, b_rb1a, W_rb1b, b_rb1b,
        W_fbs, b_fbs, W_ra1a, b_ra1a, W_ra1b, b_ra1b, W_ra2a, b_ra2a,
        W_ra2b, b_ra2b):
    B = 1280
    grid = (N_EDGES // B,)
    full = lambda shape: pl.BlockSpec(shape, lambda i: (0, 0))
    row = lambda b: b.reshape(1, EMBED)
    return pl.pallas_call(
        _k3_body,
        grid=grid,
        in_specs=[
            pl.BlockSpec((B, EMBED), lambda i: (i, 0)),
            pl.BlockSpec((B, 128), lambda i: (i, 0)),
            full(W_up_pad.shape), full(W_ji.shape), full((1, EMBED)),
            full(W_rb1a.shape), full((1, EMBED)), full(W_rb1b.shape), full((1, EMBED)),
            full(W_fbs.shape), full((1, EMBED)),
            full(W_ra1a.shape), full((1, EMBED)), full(W_ra1b.shape), full((1, EMBED)),
            full(W_ra2a.shape), full((1, EMBED)), full(W_ra2b.shape), full((1, EMBED)),
        ],
        out_specs=pl.BlockSpec((B, EMBED), lambda i: (i, 0)),
        out_shape=jax.ShapeDtypeStruct((N_EDGES, EMBED), jnp.float32),
    )(m_input, agg_pad, W_up_pad, W_ji, row(b_ji),
      W_rb1a, row(b_rb1a), W_rb1b, row(b_rb1b),
      W_fbs, row(b_fbs),
      W_ra1a, row(b_ra1a), W_ra1b, row(b_ra1b),
      W_ra2a, row(b_ra2a), W_ra2b, row(b_ra2b))


def kernel(m_input, rbf, sbf, triplet_ids, reduce_to_ji, expand_to_kj,
           W_rbf1, W_rbf2, W_sbf1, W_sbf2, W_kj, b_kj, W_down, W_up,
           W_ji, b_ji, W_rb1a, b_rb1a, W_rb1b, b_rb1b, W_fbs, b_fbs,
           W_ra1a, b_ra1a, W_ra1b, b_ra1b, W_ra2a, b_ra2a, W_ra2b, b_ra2b):
    W_rbf = W_rbf1 @ W_rbf2
    W_sbf_pad = jnp.pad(W_sbf1 @ W_sbf2, ((0, 0), (0, 64)))
    W_down_pad = jnp.pad(W_down, ((0, 0), (0, 64)))
    W_up_pad = jnp.pad(W_up, ((0, 64), (0, 0)))

    m_ang_pad = _k1(m_input, rbf.T, W_kj, b_kj, W_rbf, W_down_pad)
    sb_pad = _k2(sbf.T, W_sbf_pad)

    sv_start = lax.slice(reduce_to_ji, (0,), (N_TRIPLETS,), (T_BATCH,))
    sv_end = lax.slice(reduce_to_ji, (T_BATCH - 1,), (N_TRIPLETS,), (T_BATCH,))
    e0s = jnp.arange(NCH, dtype=jnp.int32) * E_CHUNK
    lo = jnp.sum(sv_end[None, :] < e0s[:, None], axis=1, dtype=jnp.int32)
    hi = jnp.sum(sv_start[None, :] < (e0s[:, None] + E_CHUNK), axis=1,
                 dtype=jnp.int32)
    t0 = lo * T_BATCH
    nb = hi - lo
    pack = lambda v: jnp.pad(
        jnp.pad(v, (0, NW * CH_PER_W - NCH)).reshape(NW, CH_PER_W),
        ((0, 0), (0, 32 - CH_PER_W))).reshape(-1)

    agg_pad = _sc_segment(m_ang_pad, sb_pad, expand_to_kj, reduce_to_ji,
                          pack(t0), pack(nb))

    return _k3(m_input, agg_pad, W_up_pad, W_ji, b_ji, W_rb1a, b_rb1a,
               W_rb1b, b_rb1b, W_fbs, b_fbs, W_ra1a, b_ra1a, W_ra1b, b_ra1b,
               W_ra2a, b_ra2a, W_ra2b, b_ra2b)


# speedup vs baseline: 2.1611x; 1.0813x over previous
import jax
import jax.numpy as jnp
from jax import lax
from jax.experimental import pallas as pl
from jax.experimental.pallas import tpu as pltpu
from jax.experimental.pallas import tpu_sc as plsc

N_EDGES = 320000
N_TRIPLETS = 960000
EMBED = 128
ANGLE = 64

NC = 2
NS = 16
NW = NC * NS

E_CHUNK = 320
NCH = N_EDGES // E_CHUNK
NCH_H = NCH // 2
CH_PER_W = 16
T_BATCH = 128
NB_T = N_TRIPLETS // T_BATCH


def _swish(x):
    return x * (0.5 * jnp.tanh(0.5 * x) + 0.5)


def _k1_body(m_ref, rbft_ref, wkj_ref, bkj_ref, wr_ref, wd_ref, out_ref):
    x = m_ref[...]
    h = jnp.dot(x, wkj_ref[...], preferred_element_type=jnp.float32) + bkj_ref[...]
    h = _swish(h)
    r = lax.dot_general(rbft_ref[...], wr_ref[...],
                        (((0,), (0,)), ((), ())),
                        preferred_element_type=jnp.float32)
    h = h * r
    g = jnp.dot(h, wd_ref[...], preferred_element_type=jnp.float32)
    out_ref[...] = _swish(g)


def _k1(m_input, rbfT, W_kj, b_kj, W_rbf, W_down_pad):
    B = 1280
    grid = (N_EDGES // B,)
    full = lambda shape: pl.BlockSpec(shape, lambda i: (0, 0))
    return pl.pallas_call(
        _k1_body,
        grid=grid,
        in_specs=[
            pl.BlockSpec((B, EMBED), lambda i: (i, 0)),
            pl.BlockSpec((6, B), lambda i: (0, i)),
            full(W_kj.shape),
            full((1, EMBED)),
            full(W_rbf.shape),
            full(W_down_pad.shape),
        ],
        out_specs=pl.BlockSpec((B, EMBED), lambda i: (i, 0)),
        out_shape=jax.ShapeDtypeStruct((N_EDGES, EMBED), jnp.float32),
    )(m_input, rbfT, W_kj, b_kj.reshape(1, EMBED), W_rbf, W_down_pad)


def _k2_body(sbft_ref, w_ref, out_ref):
    out_ref[...] = lax.dot_general(sbft_ref[...], w_ref[...],
                                   (((0,), (0,)), ((), ())),
                                   preferred_element_type=jnp.float32)


def _k2(sbfT, W_sbf_pad):
    B = 1920
    grid = (N_TRIPLETS // B,)
    return pl.pallas_call(
        _k2_body,
        grid=grid,
        in_specs=[
            pl.BlockSpec((42, B), lambda i: (0, i)),
            pl.BlockSpec(W_sbf_pad.shape, lambda i: (0, 0)),
        ],
        out_specs=pl.BlockSpec((B, 128), lambda i: (i, 0)),
        out_shape=jax.ShapeDtypeStruct((N_TRIPLETS, 128), jnp.float32),
    )(sbfT, W_sbf_pad)


def _sc_body(c0, m_ang_hbm, sb_hbm, expand_hbm, reduce_hbm, t0_hbm, nb_hbm,
             agg_hbm, acc_v,
             idx_e_a, idx_r_a, rows_a, sbv_a,
             idx_e_b, idx_r_b, rows_b, sbv_b,
             t0_v, nb_v, t0_s, nb_s, sem_la, sem_lb, sem_ga, sem_gb):
    wid = lax.axis_index("s") * NC + lax.axis_index("c")
    pltpu.sync_copy(t0_hbm.at[pl.ds(wid * 32, 32)], t0_v)
    pltpu.sync_copy(nb_hbm.at[pl.ds(wid * 32, 32)], nb_v)
    zeros16 = jnp.zeros((16,), jnp.float32)
    for g in range(2):
        tv = t0_v[pl.ds(g * 16, 16)]
        nv = nb_v[pl.ds(g * 16, 16)]
        for j in range(16):
            t0_s[g * 16 + j] = tv[j]
            nb_s[g * 16 + j] = nv[j]

    bufs = ((idx_e_a, idx_r_a, rows_a, sbv_a, sem_la, sem_ga),
            (idx_e_b, idx_r_b, rows_b, sbv_b, sem_lb, sem_gb))

    def issue_loads(t, bi):
        idx_e, idx_r, _, sbv, sem_l, _ = bufs[bi]
        pltpu.async_copy(expand_hbm.at[pl.ds(t, T_BATCH)], idx_e, sem_l)
        pltpu.async_copy(reduce_hbm.at[pl.ds(t, T_BATCH)], idx_r, sem_l)
        pltpu.async_copy(sb_hbm.at[pl.ds(t, T_BATCH)], sbv, sem_l)

    def wait_loads(bi):
        idx_e, idx_r, _, sbv, sem_l, _ = bufs[bi]
        pltpu.make_async_copy(expand_hbm.at[pl.ds(0, T_BATCH)], idx_e, sem_l).wait()
        pltpu.make_async_copy(reduce_hbm.at[pl.ds(0, T_BATCH)], idx_r, sem_l).wait()
        pltpu.make_async_copy(sb_hbm.at[pl.ds(0, T_BATCH)], sbv, sem_l).wait()

    def issue_gather(bi):
        idx_e, _, rows, _, _, sem_g = bufs[bi]
        pltpu.async_copy(m_ang_hbm.at[idx_e], rows, sem_g)

    def wait_gather(bi):
        idx_e, _, rows, _, _, sem_g = bufs[bi]
        pltpu.make_async_copy(m_ang_hbm.at[idx_e], rows, sem_g).wait()

    def chunk_body(ci, _carry):
        c = wid * CH_PER_W + ci
        e0 = (c0 + c) * E_CHUNK
        t0c = t0_s[ci]
        nbc = nb_s[ci]

        def zero_body(r, _):
            for f in range(8):
                acc_v[r, pl.ds(16 * f, 16)] = zeros16
            return 0
        lax.fori_loop(0, E_CHUNK + 1, zero_body, 0)

        def tt(k):
            return pl.multiple_of(t0c + k * T_BATCH, T_BATCH)

        def compute(bi):
            idx_e, idx_r, rows, sbv, _, _ = bufs[bi]

            def group_body(g, _):
                rv = idx_r[pl.ds(g * 16, 16)] - e0
                okv = jnp.logical_and(rv >= 0, rv < E_CHUNK)
                rowv = jnp.where(okv, rv, E_CHUNK)
                for j in range(16):
                    jj = g * 16 + j
                    row = rowv[j]
                    for f in range(ANGLE // 16):
                        v = rows[jj, pl.ds(16 * f, 16)] * sbv[jj, pl.ds(16 * f, 16)]
                        plsc.addupdate(acc_v.at[row, pl.ds(16 * f, 16)], v)
                return 0
            lax.fori_loop(0, T_BATCH // 16, group_body, 0)

        @pl.when(nbc > 0)
        def _prologue():
            issue_loads(tt(0), 0)

        def pair_body(m, _):
            k0 = 2 * m
            k1 = k0 + 1

            @pl.when(k0 < nbc)
            def _half0():
                wait_loads(0)
                issue_gather(0)

                @pl.when(k0 > 0)
                def _fin_prev():
                    wait_gather(1)
                    compute(1)

                @pl.when(k1 < nbc)
                def _next():
                    issue_loads(tt(k1), 1)

            @pl.when(k1 < nbc)
            def _half1():
                wait_loads(1)
                issue_gather(1)
                wait_gather(0)
                compute(0)

                @pl.when(k1 + 1 < nbc)
                def _next():
                    issue_loads(tt(k1 + 1), 0)
            return 0
        lax.fori_loop(0, (nbc + 1) // 2, pair_body, 0)

        @pl.when(jnp.logical_and(nbc > 0, (nbc & 1) == 1))
        def _epi_even():
            wait_gather(0)
            compute(0)

        @pl.when(jnp.logical_and(nbc > 0, (nbc & 1) == 0))
        def _epi_odd():
            wait_gather(1)
            compute(1)

        @pl.when(c < NCH_H)
        def _flush():
            pltpu.sync_copy(acc_v.at[pl.ds(0, E_CHUNK)],
                            agg_hbm.at[pl.ds(c * E_CHUNK, E_CHUNK)])
        return 0

    lax.fori_loop(0, CH_PER_W, chunk_body, 0)


def _sc_segment(c0, m_ang_pad, sb_pad, expand_to_kj, reduce_to_ji, t0m, nbm):
    import functools
    mesh = plsc.VectorSubcoreMesh(core_axis_name="c", subcore_axis_name="s",
                                  num_cores=NC, num_subcores=NS)
    f = pl.kernel(
        functools.partial(_sc_body, c0),
        out_type=jax.ShapeDtypeStruct((NCH_H * E_CHUNK, 128), jnp.float32),
        mesh=mesh,
        scratch_types=[
            pltpu.VMEM((E_CHUNK + 1, 128), jnp.float32),
            pltpu.VMEM((T_BATCH,), jnp.int32),
            pltpu.VMEM((T_BATCH,), jnp.int32),
            pltpu.VMEM((T_BATCH, 128), jnp.float32),
            pltpu.VMEM((T_BATCH, 128), jnp.float32),
            pltpu.VMEM((T_BATCH,), jnp.int32),
            pltpu.VMEM((T_BATCH,), jnp.int32),
            pltpu.VMEM((T_BATCH, 128), jnp.float32),
            pltpu.VMEM((T_BATCH, 128), jnp.float32),
            pltpu.VMEM((32,), jnp.int32),
            pltpu.VMEM((32,), jnp.int32),
            pltpu.SMEM((32,), jnp.int32),
            pltpu.SMEM((32,), jnp.int32),
            pltpu.SemaphoreType.DMA,
            pltpu.SemaphoreType.DMA,
            pltpu.SemaphoreType.DMA,
            pltpu.SemaphoreType.DMA,
        ],
        compiler_params=pltpu.CompilerParams(use_tc_tiling_on_sc=True),
    )
    return f(m_ang_pad, sb_pad, expand_to_kj, reduce_to_ji, t0m, nbm)


def _k3_body(m_ref, agg_ref, wup_ref, wji_ref, bji_ref,
             wrb1a_ref, brb1a_ref, wrb1b_ref, brb1b_ref,
             wfbs_ref, bfbs_ref,
             wra1a_ref, bra1a_ref, wra1b_ref, bra1b_ref,
             wra2a_ref, bra2a_ref, wra2b_ref, bra2b_ref, out_ref):
    dot = lambda a, b: jnp.dot(a, b, preferred_element_type=jnp.float32)
    x = m_ref[...]
    prop = _swish(dot(agg_ref[...], wup_ref[...]))
    m_ji = _swish(dot(x, wji_ref[...]) + bji_ref[...])
    mc = m_ji + prop
    t = _swish(dot(mc, wrb1a_ref[...]) + brb1a_ref[...])
    mc = mc + _swish(dot(t, wrb1b_ref[...]) + brb1b_ref[...])
    mc = _swish(dot(mc, wfbs_ref[...]) + bfbs_ref[...])
    out = mc + x
    t = _swish(dot(out, wra1a_ref[...]) + bra1a_ref[...])
    out = out + _swish(dot(t, wra1b_ref[...]) + bra1b_ref[...])
    t = _swish(dot(out, wra2a_ref[...]) + bra2a_ref[...])
    out_ref[...] = out + _swish(dot(t, wra2b_ref[...]) + bra2b_ref[...])


def _k3_half(off_blocks, m_input, agg_half, alias_out, W_up_pad, W_ji, b_ji,
             W_rb1a, b_rb1a, W_rb1b, b_rb1b, W_fbs, b_fbs,
             W_ra1a, b_ra1a, W_ra1b, b_ra1b, W_ra2a, b_ra2a, W_ra2b, b_ra2b):
    B = 1280
    grid = (N_EDGES // B // 2,)
    full = lambda shape: pl.BlockSpec(shape, lambda i: (0, 0))
    row = lambda b: b.reshape(1, EMBED)
    off = lambda i: (i + off_blocks, 0)
    args = [m_input, agg_half, W_up_pad, W_ji, row(b_ji),
            W_rb1a, row(b_rb1a), W_rb1b, row(b_rb1b),
            W_fbs, row(b_fbs),
            W_ra1a, row(b_ra1a), W_ra1b, row(b_ra1b),
            W_ra2a, row(b_ra2a), W_ra2b, row(b_ra2b)]
    in_specs = [
        pl.BlockSpec((B, EMBED), off),
        pl.BlockSpec((B, 128), lambda i: (i, 0)),
        full(W_up_pad.shape), full(W_ji.shape), full((1, EMBED)),
        full(W_rb1a.shape), full((1, EMBED)), full(W_rb1b.shape), full((1, EMBED)),
        full(W_fbs.shape), full((1, EMBED)),
        full(W_ra1a.shape), full((1, EMBED)), full(W_ra1b.shape), full((1, EMBED)),
        full(W_ra2a.shape), full((1, EMBED)), full(W_ra2b.shape), full((1, EMBED)),
    ]
    kw = {}
    body = _k3_body
    if alias_out is not None:
        args.append(alias_out)
        in_specs.append(pl.BlockSpec((8, EMBED), lambda i: (0, 0)))
        kw["input_output_aliases"] = {len(args) - 1: 0}
        body = lambda *refs: _k3_body(*refs[:-2], refs[-1])
    return pl.pallas_call(
        body,
        grid=grid,
        in_specs=in_specs,
        out_specs=pl.BlockSpec((B, EMBED), off),
        out_shape=jax.ShapeDtypeStruct((N_EDGES, EMBED), jnp.float32),
        **kw,
    )(*args)


def kernel(m_input, rbf, sbf, triplet_ids, reduce_to_ji, expand_to_kj,
           W_rbf1, W_rbf2, W_sbf1, W_sbf2, W_kj, b_kj, W_down, W_up,
           W_ji, b_ji, W_rb1a, b_rb1a, W_rb1b, b_rb1b, W_fbs, b_fbs,
           W_ra1a, b_ra1a, W_ra1b, b_ra1b, W_ra2a, b_ra2a, W_ra2b, b_ra2b):
    W_rbf = W_rbf1 @ W_rbf2
    W_sbf_pad = jnp.pad(W_sbf1 @ W_sbf2, ((0, 0), (0, 64)))
    W_down_pad = jnp.pad(W_down, ((0, 0), (0, 64)))
    W_up_pad = jnp.pad(W_up, ((0, 64), (0, 0)))

    m_ang_pad = _k1(m_input, rbf.T, W_kj, b_kj, W_rbf, W_down_pad)
    sb_pad = _k2(sbf.T, W_sbf_pad)

    sv_start = lax.slice(reduce_to_ji, (0,), (N_TRIPLETS,), (T_BATCH,))
    sv_end = lax.slice(reduce_to_ji, (T_BATCH - 1,), (N_TRIPLETS,), (T_BATCH,))
    e0s = jnp.arange(NCH, dtype=jnp.int32) * E_CHUNK
    lo = jnp.sum(sv_end[None, :] < e0s[:, None], axis=1, dtype=jnp.int32)
    hi = jnp.sum(sv_start[None, :] < (e0s[:, None] + E_CHUNK), axis=1,
                 dtype=jnp.int32)
    t0 = lo * T_BATCH
    nb = hi - lo
    pack = lambda v: jnp.pad(
        jnp.pad(v, (0, NW * CH_PER_W - NCH_H)).reshape(NW, CH_PER_W),
        ((0, 0), (0, 32 - CH_PER_W))).reshape(-1)

    agg1 = _sc_segment(0, m_ang_pad, sb_pad, expand_to_kj, reduce_to_ji,
                       pack(t0[:NCH_H]), pack(nb[:NCH_H]))
    agg2 = _sc_segment(NCH_H, m_ang_pad, sb_pad, expand_to_kj, reduce_to_ji,
                       pack(t0[NCH_H:]), pack(nb[NCH_H:]))

    dense = (W_up_pad, W_ji, b_ji, W_rb1a, b_rb1a, W_rb1b, b_rb1b,
             W_fbs, b_fbs, W_ra1a, b_ra1a, W_ra1b, b_ra1b,
             W_ra2a, b_ra2a, W_ra2b, b_ra2b)
    out = _k3_half(0, m_input, agg1, None, *dense)
    return _k3_half(N_EDGES // 1280 // 2, m_input, agg2, out, *dense)


# speedup vs baseline: 2.2451x; 1.0389x over previous
import jax
import jax.numpy as jnp
from jax import lax
from jax.experimental import pallas as pl
from jax.experimental.pallas import tpu as pltpu
from jax.experimental.pallas import tpu_sc as plsc

N_EDGES = 320000
N_TRIPLETS = 960000
EMBED = 128
ANGLE = 64

NC = 2
NS = 16
NW = NC * NS

E_CHUNK = 320
NCH = N_EDGES // E_CHUNK
NSPLIT = 4
NCH_Q = NCH // NSPLIT
CH_PER_W = 8
T_BATCH = 128
NB_T = N_TRIPLETS // T_BATCH


def _swish(x):
    return x * (0.5 * jnp.tanh(0.5 * x) + 0.5)


def _k1_body(m_ref, rbft_ref, wkj_ref, bkj_ref, wr_ref, wd_ref, out_ref):
    x = m_ref[...]
    h = jnp.dot(x, wkj_ref[...], preferred_element_type=jnp.float32) + bkj_ref[...]
    h = _swish(h)
    r = lax.dot_general(rbft_ref[...], wr_ref[...],
                        (((0,), (0,)), ((), ())),
                        preferred_element_type=jnp.float32)
    h = h * r
    g = jnp.dot(h, wd_ref[...], preferred_element_type=jnp.float32)
    out_ref[...] = _swish(g)


def _k1(m_input, rbfT, W_kj, b_kj, W_rbf, W_down_pad):
    B = 1280
    grid = (N_EDGES // B,)
    full = lambda shape: pl.BlockSpec(shape, lambda i: (0, 0))
    return pl.pallas_call(
        _k1_body,
        grid=grid,
        in_specs=[
            pl.BlockSpec((B, EMBED), lambda i: (i, 0)),
            pl.BlockSpec((6, B), lambda i: (0, i)),
            full(W_kj.shape),
            full((1, EMBED)),
            full(W_rbf.shape),
            full(W_down_pad.shape),
        ],
        out_specs=pl.BlockSpec((B, EMBED), lambda i: (i, 0)),
        out_shape=jax.ShapeDtypeStruct((N_EDGES, EMBED), jnp.float32),
    )(m_input, rbfT, W_kj, b_kj.reshape(1, EMBED), W_rbf, W_down_pad)


def _k2_body(sbft_ref, w_ref, out_ref):
    out_ref[...] = lax.dot_general(sbft_ref[...], w_ref[...],
                                   (((0,), (0,)), ((), ())),
                                   preferred_element_type=jnp.float32)


def _k2(sbfT, W_sbf_pad):
    B = 1920
    grid = (N_TRIPLETS // B,)
    return pl.pallas_call(
        _k2_body,
        grid=grid,
        in_specs=[
            pl.BlockSpec((42, B), lambda i: (0, i)),
            pl.BlockSpec(W_sbf_pad.shape, lambda i: (0, 0)),
        ],
        out_specs=pl.BlockSpec((B, 128), lambda i: (i, 0)),
        out_shape=jax.ShapeDtypeStruct((N_TRIPLETS, 128), jnp.float32),
    )(sbfT, W_sbf_pad)


def _sc_body(c0, m_ang_hbm, sb_hbm, expand_hbm, reduce_hbm, t0_hbm, nb_hbm,
             agg_hbm, acc_v,
             idx_e_a, idx_r_a, rows_a, sbv_a,
             idx_e_b, idx_r_b, rows_b, sbv_b,
             t0_v, nb_v, t0_s, nb_s, sem_la, sem_lb, sem_ga, sem_gb):
    wid = lax.axis_index("s") * NC + lax.axis_index("c")
    pltpu.sync_copy(t0_hbm.at[pl.ds(wid * 32, 32)], t0_v)
    pltpu.sync_copy(nb_hbm.at[pl.ds(wid * 32, 32)], nb_v)
    zeros16 = jnp.zeros((16,), jnp.float32)
    for g in range(2):
        tv = t0_v[pl.ds(g * 16, 16)]
        nv = nb_v[pl.ds(g * 16, 16)]
        for j in range(16):
            t0_s[g * 16 + j] = tv[j]
            nb_s[g * 16 + j] = nv[j]

    bufs = ((idx_e_a, idx_r_a, rows_a, sbv_a, sem_la, sem_ga),
            (idx_e_b, idx_r_b, rows_b, sbv_b, sem_lb, sem_gb))

    def issue_loads(t, bi):
        idx_e, idx_r, _, sbv, sem_l, _ = bufs[bi]
        pltpu.async_copy(expand_hbm.at[pl.ds(t, T_BATCH)], idx_e, sem_l)
        pltpu.async_copy(reduce_hbm.at[pl.ds(t, T_BATCH)], idx_r, sem_l)
        pltpu.async_copy(sb_hbm.at[pl.ds(t, T_BATCH)], sbv, sem_l)

    def wait_loads(bi):
        idx_e, idx_r, _, sbv, sem_l, _ = bufs[bi]
        pltpu.make_async_copy(expand_hbm.at[pl.ds(0, T_BATCH)], idx_e, sem_l).wait()
        pltpu.make_async_copy(reduce_hbm.at[pl.ds(0, T_BATCH)], idx_r, sem_l).wait()
        pltpu.make_async_copy(sb_hbm.at[pl.ds(0, T_BATCH)], sbv, sem_l).wait()

    def issue_gather(bi):
        idx_e, _, rows, _, _, sem_g = bufs[bi]
        pltpu.async_copy(m_ang_hbm.at[idx_e], rows, sem_g)

    def wait_gather(bi):
        idx_e, _, rows, _, _, sem_g = bufs[bi]
        pltpu.make_async_copy(m_ang_hbm.at[idx_e], rows, sem_g).wait()

    def chunk_body(ci, _carry):
        c = wid * CH_PER_W + ci
        e0 = (c0 + c) * E_CHUNK
        t0c = t0_s[ci]
        nbc = nb_s[ci]

        def zero_body(r, _):
            for f in range(8):
                acc_v[r, pl.ds(16 * f, 16)] = zeros16
            return 0
        lax.fori_loop(0, E_CHUNK + 1, zero_body, 0)

        def tt(k):
            return pl.multiple_of(t0c + k * T_BATCH, T_BATCH)

        def compute(bi):
            idx_e, idx_r, rows, sbv, _, _ = bufs[bi]

            def group_body(g, _):
                rv = idx_r[pl.ds(g * 16, 16)] - e0
                okv = jnp.logical_and(rv >= 0, rv < E_CHUNK)
                rowv = jnp.where(okv, rv, E_CHUNK)
                for j in range(16):
                    jj = g * 16 + j
                    row = rowv[j]
                    for f in range(ANGLE // 16):
                        v = rows[jj, pl.ds(16 * f, 16)] * sbv[jj, pl.ds(16 * f, 16)]
                        plsc.addupdate(acc_v.at[row, pl.ds(16 * f, 16)], v)
                return 0
            lax.fori_loop(0, T_BATCH // 16, group_body, 0)

        @pl.when(nbc > 0)
        def _prologue():
            issue_loads(tt(0), 0)

        def pair_body(m, _):
            k0 = 2 * m
            k1 = k0 + 1

            @pl.when(k0 < nbc)
            def _half0():
                wait_loads(0)
                issue_gather(0)

                @pl.when(k0 > 0)
                def _fin_prev():
                    wait_gather(1)
                    compute(1)

                @pl.when(k1 < nbc)
                def _next():
                    issue_loads(tt(k1), 1)

            @pl.when(k1 < nbc)
            def _half1():
                wait_loads(1)
                issue_gather(1)
                wait_gather(0)
                compute(0)

                @pl.when(k1 + 1 < nbc)
                def _next():
                    issue_loads(tt(k1 + 1), 0)
            return 0
        lax.fori_loop(0, (nbc + 1) // 2, pair_body, 0)

        @pl.when(jnp.logical_and(nbc > 0, (nbc & 1) == 1))
        def _epi_even():
            wait_gather(0)
            compute(0)

        @pl.when(jnp.logical_and(nbc > 0, (nbc & 1) == 0))
        def _epi_odd():
            wait_gather(1)
            compute(1)

        @pl.when(c < NCH_Q)
        def _flush():
            pltpu.sync_copy(acc_v.at[pl.ds(0, E_CHUNK)],
                            agg_hbm.at[pl.ds(c * E_CHUNK, E_CHUNK)])
        return 0

    lax.fori_loop(0, CH_PER_W, chunk_body, 0)


def _sc_segment(c0, m_ang_pad, sb_pad, expand_to_kj, reduce_to_ji, t0m, nbm):
    import functools
    mesh = plsc.VectorSubcoreMesh(core_axis_name="c", subcore_axis_name="s",
                                  num_cores=NC, num_subcores=NS)
    f = pl.kernel(
        functools.partial(_sc_body, c0),
        out_type=jax.ShapeDtypeStruct((NCH_Q * E_CHUNK, 128), jnp.float32),
        mesh=mesh,
        scratch_types=[
            pltpu.VMEM((E_CHUNK + 1, 128), jnp.float32),
            pltpu.VMEM((T_BATCH,), jnp.int32),
            pltpu.VMEM((T_BATCH,), jnp.int32),
            pltpu.VMEM((T_BATCH, 128), jnp.float32),
            pltpu.VMEM((T_BATCH, 128), jnp.float32),
            pltpu.VMEM((T_BATCH,), jnp.int32),
            pltpu.VMEM((T_BATCH,), jnp.int32),
            pltpu.VMEM((T_BATCH, 128), jnp.float32),
            pltpu.VMEM((T_BATCH, 128), jnp.float32),
            pltpu.VMEM((32,), jnp.int32),
            pltpu.VMEM((32,), jnp.int32),
            pltpu.SMEM((32,), jnp.int32),
            pltpu.SMEM((32,), jnp.int32),
            pltpu.SemaphoreType.DMA,
            pltpu.SemaphoreType.DMA,
            pltpu.SemaphoreType.DMA,
            pltpu.SemaphoreType.DMA,
        ],
        compiler_params=pltpu.CompilerParams(use_tc_tiling_on_sc=True),
    )
    return f(m_ang_pad, sb_pad, expand_to_kj, reduce_to_ji, t0m, nbm)


def _k3_body(m_ref, agg_ref, wup_ref, wji_ref, bji_ref,
             wrb1a_ref, brb1a_ref, wrb1b_ref, brb1b_ref,
             wfbs_ref, bfbs_ref,
             wra1a_ref, bra1a_ref, wra1b_ref, bra1b_ref,
             wra2a_ref, bra2a_ref, wra2b_ref, bra2b_ref, out_ref):
    dot = lambda a, b: jnp.dot(a, b, preferred_element_type=jnp.float32)
    x = m_ref[...]
    prop = _swish(dot(agg_ref[...], wup_ref[...]))
    m_ji = _swish(dot(x, wji_ref[...]) + bji_ref[...])
    mc = m_ji + prop
    t = _swish(dot(mc, wrb1a_ref[...]) + brb1a_ref[...])
    mc = mc + _swish(dot(t, wrb1b_ref[...]) + brb1b_ref[...])
    mc = _swish(dot(mc, wfbs_ref[...]) + bfbs_ref[...])
    out = mc + x
    t = _swish(dot(out, wra1a_ref[...]) + bra1a_ref[...])
    out = out + _swish(dot(t, wra1b_ref[...]) + bra1b_ref[...])
    t = _swish(dot(out, wra2a_ref[...]) + bra2a_ref[...])
    out_ref[...] = out + _swish(dot(t, wra2b_ref[...]) + bra2b_ref[...])


def _k3_half(off_blocks, m_input, agg_half, alias_out, W_up_pad, W_ji, b_ji,
             W_rb1a, b_rb1a, W_rb1b, b_rb1b, W_fbs, b_fbs,
             W_ra1a, b_ra1a, W_ra1b, b_ra1b, W_ra2a, b_ra2a, W_ra2b, b_ra2b):
    B = 1600
    grid = (N_EDGES // B // NSPLIT,)
    full = lambda shape: pl.BlockSpec(shape, lambda i: (0, 0))
    row = lambda b: b.reshape(1, EMBED)
    off = lambda i: (i + off_blocks, 0)
    args = [m_input, agg_half, W_up_pad, W_ji, row(b_ji),
            W_rb1a, row(b_rb1a), W_rb1b, row(b_rb1b),
            W_fbs, row(b_fbs),
            W_ra1a, row(b_ra1a), W_ra1b, row(b_ra1b),
            W_ra2a, row(b_ra2a), W_ra2b, row(b_ra2b)]
    in_specs = [
        pl.BlockSpec((B, EMBED), off),
        pl.BlockSpec((B, 128), lambda i: (i, 0)),
        full(W_up_pad.shape), full(W_ji.shape), full((1, EMBED)),
        full(W_rb1a.shape), full((1, EMBED)), full(W_rb1b.shape), full((1, EMBED)),
        full(W_fbs.shape), full((1, EMBED)),
        full(W_ra1a.shape), full((1, EMBED)), full(W_ra1b.shape), full((1, EMBED)),
        full(W_ra2a.shape), full((1, EMBED)), full(W_ra2b.shape), full((1, EMBED)),
    ]
    kw = {}
    body = _k3_body
    if alias_out is not None:
        args.append(alias_out)
        in_specs.append(pl.BlockSpec((8, EMBED), lambda i: (0, 0)))
        kw["input_output_aliases"] = {len(args) - 1: 0}
        body = lambda *refs: _k3_body(*refs[:-2], refs[-1])
    return pl.pallas_call(
        body,
        grid=grid,
        in_specs=in_specs,
        out_specs=pl.BlockSpec((B, EMBED), off),
        out_shape=jax.ShapeDtypeStruct((N_EDGES, EMBED), jnp.float32),
        **kw,
    )(*args)


def kernel(m_input, rbf, sbf, triplet_ids, reduce_to_ji, expand_to_kj,
           W_rbf1, W_rbf2, W_sbf1, W_sbf2, W_kj, b_kj, W_down, W_up,
           W_ji, b_ji, W_rb1a, b_rb1a, W_rb1b, b_rb1b, W_fbs, b_fbs,
           W_ra1a, b_ra1a, W_ra1b, b_ra1b, W_ra2a, b_ra2a, W_ra2b, b_ra2b):
    W_rbf = W_rbf1 @ W_rbf2
    W_sbf_pad = jnp.pad(W_sbf1 @ W_sbf2, ((0, 0), (0, 64)))
    W_down_pad = jnp.pad(W_down, ((0, 0), (0, 64)))
    W_up_pad = jnp.pad(W_up, ((0, 64), (0, 0)))

    m_ang_pad = _k1(m_input, rbf.T, W_kj, b_kj, W_rbf, W_down_pad)
    sb_pad = _k2(sbf.T, W_sbf_pad)

    sv_start = lax.slice(reduce_to_ji, (0,), (N_TRIPLETS,), (T_BATCH,))
    sv_end = lax.slice(reduce_to_ji, (T_BATCH - 1,), (N_TRIPLETS,), (T_BATCH,))
    e0s = jnp.arange(NCH, dtype=jnp.int32) * E_CHUNK
    lo = jnp.sum(sv_end[None, :] < e0s[:, None], axis=1, dtype=jnp.int32)
    hi = jnp.sum(sv_start[None, :] < (e0s[:, None] + E_CHUNK), axis=1,
                 dtype=jnp.int32)
    t0 = lo * T_BATCH
    nb = hi - lo
    pack = lambda v: jnp.pad(
        jnp.pad(v, (0, NW * CH_PER_W - NCH_Q)).reshape(NW, CH_PER_W),
        ((0, 0), (0, 32 - CH_PER_W))).reshape(-1)

    aggs = [
        _sc_segment(q * NCH_Q, m_ang_pad, sb_pad, expand_to_kj, reduce_to_ji,
                    pack(t0[q * NCH_Q:(q + 1) * NCH_Q]),
                    pack(nb[q * NCH_Q:(q + 1) * NCH_Q]))
        for q in range(NSPLIT)
    ]

    dense = (W_up_pad, W_ji, b_ji, W_rb1a, b_rb1a, W_rb1b, b_rb1b,
             W_fbs, b_fbs, W_ra1a, b_ra1a, W_ra1b, b_ra1b,
             W_ra2a, b_ra2a, W_ra2b, b_ra2b)
    qblocks = N_EDGES // 1600 // NSPLIT
    out = None
    for q in range(NSPLIT):
        out = _k3_half(q * qblocks, m_input, aggs[q], out, *dense)
    return out
